# baseline (speedup 1.0000x reference)
import functools

import jax
import jax.numpy as jnp
from jax import lax
from jax.experimental import pallas as pl
from jax.experimental.pallas import tpu as pltpu
from jax.experimental.pallas import tpu_sc as plsc

VOCAB = 1000000
EMB = 32
SEQ = 50

_INFO = plsc.get_sparse_core_info()
NC = _INFO.num_cores
NS = _INFO.num_subcores
NW = NC * NS

CHUNK_B = 512
TC_PER_CHUNK = CHUNK_B // 128
TBL_W = 128

_A_COLS = 2048


def _table_body(in_ref, out_ref):
  t = in_ref[...].T
  out_ref[...] = jnp.pad(t, ((0, 0), (0, TBL_W - EMB)))


def _make_table_kernel():
  grid = (VOCAB + _A_COLS - 1) // _A_COLS
  return pl.pallas_call(
      _table_body,
      out_shape=jax.ShapeDtypeStruct((VOCAB, TBL_W), jnp.float32),
      grid=(grid,),
      in_specs=[pl.BlockSpec((EMB, _A_COLS), lambda i: (0, i))],
      out_specs=pl.BlockSpec((_A_COLS, TBL_W), lambda i: (i, 0)),
  )


def _gather_body(n_chunks_pw, tj_groups, b_total, idx_hbm, table_hbm, o7_hbm,
                 idx_v, buf_v, trows_v0, trows_v1, gsem, osem):
  c = lax.axis_index("c")
  s = lax.axis_index("s")
  wid = s * NC + c
  trows_bufs = [trows_v0, trows_v1]

  def chunk_body(i, carry):
    k = wid * n_chunks_pw + i
    s_id = k // tj_groups
    tj0 = (k % tj_groups) * TC_PER_CHUNK
    off = s_id * b_total + tj0 * 128
    pltpu.sync_copy(idx_hbm.at[pl.ds(off, CHUNK_B)], idx_v)
    pltpu.async_copy(table_hbm.at[idx_v], buf_v, gsem).wait()

    lane_e = lax.iota(jnp.int32, 16) * 128
    pend = [None, None]
    for tc in range(TC_PER_CHUNK):
      tb = trows_bufs[tc % 2]
      if pend[tc % 2] is not None:
        for p in pend[tc % 2]:
          p.wait()
      for rr in range(128):
        r = tc * 128 + rr
        lo = buf_v[r, pl.ds(0, 16)]
        hi = buf_v[r, pl.ds(16, 16)]
        plsc.store_scatter(tb, [lane_e + rr], lo)
        plsc.store_scatter(tb, [lane_e + (16 * 128 + rr)], hi)
      pend[tc % 2] = [
          pltpu.async_copy(tb.at[pl.ds(ti * 1024, 1024)],
                           o7_hbm.at[s_id, ti, tj0 + tc], osem)
          for ti in range(4)
      ]
    for plist in pend:
      if plist is not None:
        for p in plist:
          p.wait()
    return carry

  lax.fori_loop(0, n_chunks_pw, chunk_body, 0)


def kernel(inputs, embeddings):
  b, seq = inputs.shape
  assert seq == SEQ and b % 128 == 0
  tj_all = b // 128
  assert tj_all % TC_PER_CHUNK == 0
  tj_groups = tj_all // TC_PER_CHUNK
  n_chunks = SEQ * tj_groups
  assert n_chunks % NW == 0
  n_chunks_pw = n_chunks // NW

  idx_sm = jnp.reshape(inputs.T, (-1,)).astype(jnp.int32)
  emb_t = embeddings.T
  tbl_pad = _make_table_kernel()(emb_t)

  mesh = plsc.VectorSubcoreMesh(core_axis_name="c", subcore_axis_name="s")
  gather = pl.kernel(
      functools.partial(_gather_body, n_chunks_pw, tj_groups, b),
      out_type=jax.ShapeDtypeStruct((SEQ, 4, tj_all, 1024), jnp.float32),
      mesh=mesh,
      scratch_types=[
          pltpu.VMEM((CHUNK_B,), jnp.int32),
          pltpu.VMEM((CHUNK_B, TBL_W), jnp.float32),
          pltpu.VMEM((EMB * 128,), jnp.float32),
          pltpu.VMEM((EMB * 128,), jnp.float32),
          pltpu.SemaphoreType.DMA,
          pltpu.SemaphoreType.DMA,
      ],
      compiler_params=pltpu.CompilerParams(use_tc_tiling_on_sc=False,
                                           needs_layout_passes=False),
  )
  o7 = gather(idx_sm, tbl_pad)
  o7b = jnp.reshape(o7, (SEQ, 4, tj_all, 8, 128))
  o5 = jnp.transpose(o7b, (0, 1, 3, 2, 4))
  o3 = jnp.reshape(o5, (SEQ, EMB, b))
  return jnp.transpose(o3, (2, 0, 1))


# speedup vs baseline: 1.1128x; 1.1128x over previous
import functools

import jax
import jax.numpy as jnp
from jax import lax
from jax.experimental import pallas as pl
from jax.experimental.pallas import tpu as pltpu
from jax.experimental.pallas import tpu_sc as plsc

EMB = 32
SEQ = 50

_INFO = plsc.get_sparse_core_info()
NC = _INFO.num_cores
NS = _INFO.num_subcores
NW = NC * NS

CHUNK_R = 32


def _gather_body(n_chunks, idx_hbm, table_hbm, out_hbm, idx_v0, idx_v1,
                 rows_v0, rows_v1, gsem, osem):
  c = lax.axis_index("c")
  s = lax.axis_index("s")
  wid = s * NC + c
  chunk_f = CHUNK_R * SEQ
  base_r = wid * n_chunks * CHUNK_R
  base_f = base_r * SEQ
  idx_bufs = [idx_v0, idx_v1]
  rows_bufs = [rows_v0, rows_v1]

  pltpu.sync_copy(idx_hbm.at[pl.ds(base_f, chunk_f)], idx_bufs[0])
  g = pltpu.async_copy(table_hbm.at[idx_bufs[0]], rows_bufs[0], gsem)
  out_pending = [[], []]
  for i in range(n_chunks):
    cur = i % 2
    nxt = 1 - cur
    if i + 1 < n_chunks:
      pltpu.sync_copy(
          idx_hbm.at[pl.ds(base_f + (i + 1) * chunk_f, chunk_f)],
          idx_bufs[nxt])
    g.wait()
    if i + 1 < n_chunks:
      for p in out_pending[nxt]:
        p.wait()
      out_pending[nxt] = []
      g = pltpu.async_copy(table_hbm.at[idx_bufs[nxt]], rows_bufs[nxt], gsem)
    out_pending[cur] = [
        pltpu.async_copy(rows_bufs[cur].at[pl.ds(r * SEQ, SEQ)],
                         out_hbm.at[base_r + i * CHUNK_R + r], osem)
        for r in range(CHUNK_R)
    ]
  for plist in out_pending:
    for p in plist:
      p.wait()


def kernel(inputs, embeddings):
  b, seq = inputs.shape
  flat_idx = jnp.reshape(inputs, (-1,)).astype(jnp.int32)
  assert seq == SEQ and b % (NW * CHUNK_R) == 0, (inputs.shape,)
  n_chunks = b // (NW * CHUNK_R)

  mesh = plsc.VectorSubcoreMesh(core_axis_name="c", subcore_axis_name="s")
  gather = pl.kernel(
      functools.partial(_gather_body, n_chunks),
      out_type=jax.ShapeDtypeStruct((b, SEQ, EMB), jnp.float32),
      mesh=mesh,
      scratch_types=[
          pltpu.VMEM((CHUNK_R * SEQ,), jnp.int32),
          pltpu.VMEM((CHUNK_R * SEQ,), jnp.int32),
          pltpu.VMEM((CHUNK_R * SEQ, EMB), jnp.float32),
          pltpu.VMEM((CHUNK_R * SEQ, EMB), jnp.float32),
          pltpu.SemaphoreType.DMA,
          pltpu.SemaphoreType.DMA,
      ],
      compiler_params=pltpu.CompilerParams(use_tc_tiling_on_sc=False),
  )
  return gather(flat_idx, embeddings)
